# baseline (speedup 1.0000x reference)
import jax
import jax.numpy as jnp
from jax.experimental import pallas as pl
from jax.experimental.pallas import tpu as pltpu

NUM_CLASSES = 100
NUM_NEIGHBOURS = 200
SIGMA = 10.0
GC = 1.0 / (2.0 * SIGMA ** 2)

BQ = 256


def _body(f_ref, c_ref, a2_ref, b2_ref, l_ref, w_ref, o_ref):
    f = f_ref[...]
    c = c_ref[...]
    a2 = a2_ref[...]
    b2 = b2_ref[...]
    m = c.shape[0]

    ab = jax.lax.dot_general(f, c, (((1,), (1,)), ((), ())))
    d2 = jnp.maximum((a2 + b2) - 2.0 * ab, 0.0)

    kf = float(NUM_NEIGHBOURS)
    mm = float(m)
    mu = jnp.sum(d2, axis=1, keepdims=True) * (1.0 / mm)
    a_bits = jnp.full(mu.shape, -1, jnp.int32)
    b_bits = jnp.full(mu.shape, 0x7F800000, jnp.int32)
    ms = jnp.sum(d2 * d2, axis=1, keepdims=True) * (1.0 / mm)
    sig = jnp.sqrt(jnp.maximum(ms - mu * mu, 1e-30))
    n_a = jnp.zeros_like(mu)
    n_b = jnp.full_like(mu, mm)
    zk = -2.2505
    dens = mm * 0.0317

    def cond(carry):
        a, b, na, nb, it = carry
        return jnp.any(a < b)

    def step(carry):
        a, b, na, nb, it = carry
        active = a < b
        a_val = jnp.where(a < 0, 0.0,
                          jax.lax.bitcast_convert_type(a, jnp.float32))
        b_val = jax.lax.bitcast_convert_type(b, jnp.float32)
        frac = (kf - na + 0.5) / (nb - na + 1.0)
        t_sec = a_val + (b_val - a_val) * frac
        use_a = (kf - na) <= (nb - kf)
        t_anchor = jnp.where(use_a, a_val, b_val)
        c_anchor = jnp.where(use_a, na, nb)
        t_newt = t_anchor + (kf - c_anchor) * sig * (1.0 / dens)
        t0f = jnp.where(nb - na <= 32.0, t_sec, t_newt)
        t0f = jnp.where(it == 0, mu + zk * sig, t0f)
        t0 = jax.lax.bitcast_convert_type(jnp.maximum(t0f, 0.0), jnp.int32)
        t0 = jnp.where(it % 6 == 5, a + (b - a) // 2, t0)
        t = jnp.maximum(jnp.minimum(t0, b - 1), a + 1)
        t_f = jax.lax.bitcast_convert_type(t, jnp.float32)
        cnt = jnp.sum((d2 <= t_f).astype(jnp.float32), axis=1, keepdims=True)
        ge = cnt >= kf
        new_b = jnp.where(ge, t, b)
        new_nb = jnp.where(ge, cnt, nb)
        new_a = jnp.where(ge, a, t)
        new_na = jnp.where(ge, na, cnt)
        new_a = jnp.where(cnt == kf, t, new_a)
        new_a = jnp.where(new_b == new_a + 1, new_b, new_a)
        return (jnp.where(active, new_a, a),
                jnp.where(active, new_b, b),
                jnp.where(active & ~ge, new_na, na),
                jnp.where(active & ge, new_nb, nb),
                it + 1)

    def step2(carry):
        return step(step(carry))

    a_bits, b_bits, n_a, n_b, _ = jax.lax.while_loop(
        cond, step2, (a_bits, b_bits, n_a, n_b, jnp.int32(0)))
    thresh = jax.lax.bitcast_convert_type(a_bits, jnp.float32)

    v = jnp.where(d2 <= thresh, jnp.exp(w_ref[...] - d2 * GC), 0.0)

    oh = (jax.lax.broadcasted_iota(jnp.int32, (NUM_CLASSES, m), 0)
          == l_ref[...]).astype(jnp.float32)
    p = jax.lax.dot_general(v, oh, (((1,), (1,)), ((), ())))
    p = jnp.where(p == 0.0, 1e-10, p)
    p = p / jnp.sum(p, axis=1, keepdims=True)
    o_ref[...] = jnp.log(p)


@jax.jit
def kernel(features, centres, centre_labels, weight):
    b, d = features.shape
    m = centres.shape[0]
    a2 = jnp.sum(features * features, axis=1)[:, None]
    b2 = jnp.sum(centres * centres, axis=1)[None, :]
    labels2 = centre_labels.reshape(1, m).astype(jnp.int32)
    weight2 = weight.reshape(1, m)

    grid = (b // BQ,)
    return pl.pallas_call(
        _body,
        grid=grid,
        in_specs=[
            pl.BlockSpec((BQ, d), lambda i: (i, 0)),
            pl.BlockSpec((m, d), lambda i: (0, 0)),
            pl.BlockSpec((BQ, 1), lambda i: (i, 0)),
            pl.BlockSpec((1, m), lambda i: (0, 0)),
            pl.BlockSpec((1, m), lambda i: (0, 0)),
            pl.BlockSpec((1, m), lambda i: (0, 0)),
        ],
        out_specs=pl.BlockSpec((BQ, NUM_CLASSES), lambda i: (i, 0)),
        out_shape=jax.ShapeDtypeStruct((b, NUM_CLASSES), jnp.float32),
        compiler_params=pltpu.CompilerParams(
            dimension_semantics=("parallel",)),
    )(features, centres, a2, b2, labels2, weight2)


# speedup vs baseline: 1.0445x; 1.0445x over previous
import jax
import jax.numpy as jnp
from jax.experimental import pallas as pl
from jax.experimental.pallas import tpu as pltpu

NUM_CLASSES = 100
NUM_NEIGHBOURS = 200
SIGMA = 10.0
GC = 1.0 / (2.0 * SIGMA ** 2)

BQ = 256


def _body(f_ref, c_ref, a2_ref, b2_ref, l_ref, w_ref, o_ref):
    f = f_ref[...]
    c = c_ref[...]
    a2 = a2_ref[...]
    b2 = b2_ref[...]
    m = c.shape[0]

    ab = jax.lax.dot_general(f, c, (((1,), (1,)), ((), ())))
    d2 = jnp.maximum((a2 + b2) - 2.0 * ab, 0.0)

    kf = float(NUM_NEIGHBOURS)
    mm = float(m)
    d2s = d2[:, :4096]
    mu = jnp.sum(d2s, axis=1, keepdims=True) * (1.0 / 4096.0)
    a_bits = jnp.full(mu.shape, -1, jnp.int32)
    b_bits = jnp.full(mu.shape, 0x7F800000, jnp.int32)
    ms = jnp.sum(d2s * d2s, axis=1, keepdims=True) * (1.0 / 4096.0)
    sig = jnp.sqrt(jnp.maximum(ms - mu * mu, 1e-30))
    n_a = jnp.zeros_like(mu)
    n_b = jnp.full_like(mu, mm)
    zk = -2.2505
    dens = mm * 0.0317

    def cond(carry):
        a, b, na, nb, it = carry
        return jnp.any(a < b)

    def step(carry):
        a, b, na, nb, it = carry
        active = a < b
        a_val = jnp.where(a < 0, 0.0,
                          jax.lax.bitcast_convert_type(a, jnp.float32))
        b_val = jax.lax.bitcast_convert_type(b, jnp.float32)
        frac = (kf - na + 0.5) / (nb - na + 1.0)
        t_sec = a_val + (b_val - a_val) * frac
        use_a = (kf - na) <= (nb - kf)
        t_anchor = jnp.where(use_a, a_val, b_val)
        c_anchor = jnp.where(use_a, na, nb)
        t_newt = t_anchor + (kf - c_anchor) * sig * (1.0 / dens)
        t0f = jnp.where(nb - na <= 32.0, t_sec, t_newt)
        t0f = jnp.where(it == 0, mu + zk * sig, t0f)
        t0 = jax.lax.bitcast_convert_type(jnp.maximum(t0f, 0.0), jnp.int32)
        t0 = jnp.where(it % 6 == 5, a + (b - a) // 2, t0)
        t = jnp.maximum(jnp.minimum(t0, b - 1), a + 1)
        t_f = jax.lax.bitcast_convert_type(t, jnp.float32)
        cnt = jnp.sum((d2 <= t_f).astype(jnp.float32), axis=1, keepdims=True)
        ge = cnt >= kf
        new_b = jnp.where(ge, t, b)
        new_nb = jnp.where(ge, cnt, nb)
        new_a = jnp.where(ge, a, t)
        new_na = jnp.where(ge, na, cnt)
        new_a = jnp.where(cnt == kf, t, new_a)
        new_a = jnp.where(new_b == new_a + 1, new_b, new_a)
        return (jnp.where(active, new_a, a),
                jnp.where(active, new_b, b),
                jnp.where(active & ~ge, new_na, na),
                jnp.where(active & ge, new_nb, nb),
                it + 1)

    a_bits, b_bits, n_a, n_b, _ = jax.lax.while_loop(
        cond, step, (a_bits, b_bits, n_a, n_b, jnp.int32(0)))
    thresh = jax.lax.bitcast_convert_type(a_bits, jnp.float32)

    v = jnp.where(d2 <= thresh, jnp.exp(w_ref[...] - d2 * GC), 0.0)

    oh = (jax.lax.broadcasted_iota(jnp.int32, (NUM_CLASSES, m), 0)
          == l_ref[...]).astype(jnp.float32)
    p = jax.lax.dot_general(v, oh, (((1,), (1,)), ((), ())))
    p = jnp.where(p == 0.0, 1e-10, p)
    p = p / jnp.sum(p, axis=1, keepdims=True)
    o_ref[...] = jnp.log(p)


@jax.jit
def kernel(features, centres, centre_labels, weight):
    b, d = features.shape
    m = centres.shape[0]
    a2 = jnp.sum(features * features, axis=1)[:, None]
    b2 = jnp.sum(centres * centres, axis=1)[None, :]
    labels2 = centre_labels.reshape(1, m).astype(jnp.int32)
    weight2 = weight.reshape(1, m)

    grid = (b // BQ,)
    return pl.pallas_call(
        _body,
        grid=grid,
        in_specs=[
            pl.BlockSpec((BQ, d), lambda i: (i, 0)),
            pl.BlockSpec((m, d), lambda i: (0, 0)),
            pl.BlockSpec((BQ, 1), lambda i: (i, 0)),
            pl.BlockSpec((1, m), lambda i: (0, 0)),
            pl.BlockSpec((1, m), lambda i: (0, 0)),
            pl.BlockSpec((1, m), lambda i: (0, 0)),
        ],
        out_specs=pl.BlockSpec((BQ, NUM_CLASSES), lambda i: (i, 0)),
        out_shape=jax.ShapeDtypeStruct((b, NUM_CLASSES), jnp.float32),
        compiler_params=pltpu.CompilerParams(
            dimension_semantics=("parallel",)),
    )(features, centres, a2, b2, labels2, weight2)
